# baseline (speedup 1.0000x reference)
import jax
import jax.numpy as jnp
from jax.experimental import pallas as pl


def _routing_kernel(x_ref, w1t_ref, b1_ref, w2t_ref, b2_ref,
                    probs_ref, hd_ref, hc_ref, aux_ref):
    x = x_ref[...]
    h = jnp.dot(x, w1t_ref[...], preferred_element_type=jnp.float32)
    h = jnp.maximum(h + b1_ref[...], 0.0)
    logits = jnp.dot(h, w2t_ref[...], preferred_element_type=jnp.float32)
    logits = logits + b2_ref[...]

    m = jnp.max(logits, axis=-1, keepdims=True)
    ex = jnp.exp(logits - m)
    probs = ex / jnp.sum(ex, axis=-1, keepdims=True)
    probs_ref[...] = probs

    S, E = probs.shape
    iota = jax.lax.broadcasted_iota(jnp.int32, (S, E), 1)
    e0 = jnp.argmax(probs, axis=-1)
    oh0 = iota == e0[:, None]
    p0 = jnp.max(probs, axis=-1, keepdims=True)
    masked = jnp.where(oh0, -1.0, probs)
    e1 = jnp.argmax(masked, axis=-1)
    oh1 = iota == e1[:, None]
    p1 = jnp.max(masked, axis=-1, keepdims=True)
    tot = p0 + p1
    p0n = p0 / tot
    p1n = p1 / tot

    a = jnp.max(oh0.astype(jnp.float32), axis=0, keepdims=True)
    s1 = jnp.sum(oh1.astype(jnp.float32) * a, axis=-1, keepdims=True)
    sec0 = jnp.logical_and(oh1, s1 == 0.0)
    sec1 = jnp.logical_and(oh1, s1 > 0.0)

    oh0f = oh0.astype(jnp.float32)
    sec0f = sec0.astype(jnp.float32)
    sec1f = sec1.astype(jnp.float32)
    d0 = oh0f + sec0f
    d1 = sec1f
    c0 = oh0f * p0n + sec0f * p1n
    c1 = sec1f * p1n
    hd_ref[...] = jnp.stack([d0, d1], axis=-1)
    hc_ref[...] = jnp.stack([c0, c1], axis=-1)

    mean_probs = jnp.mean(probs, axis=0, keepdims=True)
    aux = jnp.sum(mean_probs * jnp.log(mean_probs * E + 1e-9),
                  axis=-1, keepdims=True)
    aux_ref[...] = aux


def _expand_kernel(hd_ref, hc_ref, disp_ref, comb_ref):
    t, e, cap = disp_ref.shape
    z = jnp.zeros((t, e, cap), jnp.float32)
    disp_ref[...] = z
    comb_ref[...] = z
    disp_ref[:, :, 0:2] = hd_ref[...]
    comb_ref[:, :, 0:2] = hc_ref[...]


def kernel(hidden_states, W1, b1, W2, b2):
    B, S, H = hidden_states.shape
    E = W2.shape[0]
    k = 2
    capacity = int(B * S * 1.5 * k / E)
    N = B * S

    x = hidden_states.reshape(N, H)
    w1t = W1.T
    w2t = W2.T
    b1r = b1.reshape(1, H)
    b2r = b2.reshape(1, E)

    probs, hd, hc, aux = pl.pallas_call(
        _routing_kernel,
        out_shape=[
            jax.ShapeDtypeStruct((N, E), jnp.float32),
            jax.ShapeDtypeStruct((N, E, 2), jnp.float32),
            jax.ShapeDtypeStruct((N, E, 2), jnp.float32),
            jax.ShapeDtypeStruct((1, 1), jnp.float32),
        ],
    )(x, w1t, b1r, w2t, b2r)

    T = 128
    nblk = N // T
    head_spec = pl.BlockSpec((T, E, 2), lambda i: (i, 0, 0))
    out_spec = pl.BlockSpec((T, E, capacity), lambda i: (i, 0, 0))
    dispatch, combine = pl.pallas_call(
        _expand_kernel,
        grid=(nblk,),
        in_specs=[head_spec, head_spec],
        out_specs=[out_spec, out_spec],
        out_shape=[
            jax.ShapeDtypeStruct((N, E, capacity), jnp.float32),
            jax.ShapeDtypeStruct((N, E, capacity), jnp.float32),
        ],
    )(hd, hc)

    return (dispatch.reshape(B, S, E, capacity),
            combine.reshape(B, S, E, capacity),
            probs.reshape(B, S, E),
            aux[0, 0])


# speedup vs baseline: 1.2671x; 1.2671x over previous
import jax
import jax.numpy as jnp
from jax.experimental import pallas as pl


def _routing_kernel(x_ref, w1t_ref, b1_ref, w2t_ref, b2_ref,
                    probs_ref, d0_ref, d1_ref, c0_ref, c1_ref, aux_ref):
    x = x_ref[...]
    h = jnp.dot(x, w1t_ref[...], preferred_element_type=jnp.float32)
    h = jnp.maximum(h + b1_ref[...], 0.0)
    logits = jnp.dot(h, w2t_ref[...], preferred_element_type=jnp.float32)
    logits = logits + b2_ref[...]

    m = jnp.max(logits, axis=-1, keepdims=True)
    ex = jnp.exp(logits - m)
    probs = ex / jnp.sum(ex, axis=-1, keepdims=True)
    probs_ref[...] = probs

    S, E = probs.shape
    iota = jax.lax.broadcasted_iota(jnp.int32, (S, E), 1)
    e0 = jnp.argmax(probs, axis=-1)
    oh0 = iota == e0[:, None]
    p0 = jnp.max(probs, axis=-1, keepdims=True)
    masked = jnp.where(oh0, -1.0, probs)
    e1 = jnp.argmax(masked, axis=-1)
    oh1 = iota == e1[:, None]
    p1 = jnp.max(masked, axis=-1, keepdims=True)
    tot = p0 + p1
    p0n = p0 / tot
    p1n = p1 / tot

    a = jnp.max(oh0.astype(jnp.float32), axis=0, keepdims=True)
    s1 = jnp.sum(oh1.astype(jnp.float32) * a, axis=-1, keepdims=True)
    sec0 = jnp.logical_and(oh1, s1 == 0.0)
    sec1 = jnp.logical_and(oh1, s1 > 0.0)

    oh0f = oh0.astype(jnp.float32)
    sec0f = sec0.astype(jnp.float32)
    sec1f = sec1.astype(jnp.float32)
    d0_ref[...] = oh0f + sec0f
    d1_ref[...] = sec1f
    c0_ref[...] = oh0f * p0n + sec0f * p1n
    c1_ref[...] = sec1f * p1n

    mean_probs = jnp.mean(probs, axis=0, keepdims=True)
    aux = jnp.sum(mean_probs * jnp.log(mean_probs * E + 1e-9),
                  axis=-1, keepdims=True)
    aux_ref[...] = aux


def _expand_kernel(d0_ref, d1_ref, c0_ref, c1_ref, disp_ref, comb_ref):
    t, e, cap = disp_ref.shape
    L = 128
    i = pl.program_id(0)
    rows = pl.ds(i * t, t)
    ci = jax.lax.broadcasted_iota(jnp.int32, (t, e, L), 2)
    is0 = ci == 0
    is1 = ci == 1
    d0 = d0_ref[rows, :][:, :, None]
    d1 = d1_ref[rows, :][:, :, None]
    c0 = c0_ref[rows, :][:, :, None]
    c1 = c1_ref[rows, :][:, :, None]
    disp_ref[:, :, :L] = jnp.where(is0, d0, jnp.where(is1, d1, 0.0))
    comb_ref[:, :, :L] = jnp.where(is0, c0, jnp.where(is1, c1, 0.0))
    tail = jnp.zeros((t, e, cap - L), jnp.float32)
    disp_ref[:, :, L:] = tail
    comb_ref[:, :, L:] = tail


def kernel(hidden_states, W1, b1, W2, b2):
    B, S, H = hidden_states.shape
    E = W2.shape[0]
    k = 2
    capacity = int(B * S * 1.5 * k / E)
    N = B * S

    x = hidden_states.reshape(N, H)
    w1t = W1.T
    w2t = W2.T
    b1r = b1.reshape(1, H)
    b2r = b2.reshape(1, E)

    probs, d0, d1, c0, c1, aux = pl.pallas_call(
        _routing_kernel,
        out_shape=[
            jax.ShapeDtypeStruct((N, E), jnp.float32),
            jax.ShapeDtypeStruct((N, E), jnp.float32),
            jax.ShapeDtypeStruct((N, E), jnp.float32),
            jax.ShapeDtypeStruct((N, E), jnp.float32),
            jax.ShapeDtypeStruct((N, E), jnp.float32),
            jax.ShapeDtypeStruct((1, 1), jnp.float32),
        ],
    )(x, w1t, b1r, w2t, b2r)

    T = 128
    nblk = N // T
    plane_spec = pl.BlockSpec((N, E), lambda i: (0, 0))
    out_spec = pl.BlockSpec((T, E, capacity), lambda i: (i, 0, 0))
    dispatch, combine = pl.pallas_call(
        _expand_kernel,
        grid=(nblk,),
        in_specs=[plane_spec, plane_spec, plane_spec, plane_spec],
        out_specs=[out_spec, out_spec],
        out_shape=[
            jax.ShapeDtypeStruct((N, E, capacity), jnp.float32),
            jax.ShapeDtypeStruct((N, E, capacity), jnp.float32),
        ],
    )(d0, d1, c0, c1)

    return (dispatch.reshape(B, S, E, capacity),
            combine.reshape(B, S, E, capacity),
            probs.reshape(B, S, E),
            aux[0, 0])
